# Initial kernel scaffold; baseline (speedup 1.0000x reference)
import jax
import jax.numpy as jnp
from jax.experimental import pallas as pl


def kernel(tokens, words, positions, ln_gamma, ln_beta):
    raise NotImplementedError("write your pallas kernel here")


# speedup vs baseline: 1.3355x; 1.3355x over previous
import functools

import jax
import jax.numpy as jnp
from jax import lax
from jax.experimental import pallas as pl
from jax.experimental.pallas import tpu as pltpu
from jax.experimental.pallas import tpu_sc as plsc

VOCAB = 1000000
HIDDEN = 64
MAX_LEN = 200
SEQ = 200
EPS = 1e-08

L = 16
NW = 32
CH = 256
GSUB = 128
NVH = HIDDEN // L


def _rsqrt(a):
    i = plsc.bitcast(a, jnp.int32)
    i = jnp.int32(0x5F3759DF) - lax.shift_right_logical(i, 1)
    y = plsc.bitcast(i, jnp.float32)
    half = jnp.float32(0.5) * a
    for _ in range(3):
        y = y * (jnp.float32(1.5) - half * y * y)
    return y


def _body(tokens_hbm, words_hbm, positions_hbm, gamma_hbm, beta_hbm,
          out_hbm, idx_v, rows_v, pos_v, gb_v, gsem, osem):
    info = plsc.get_sparse_core_info()
    nc = info.num_cores
    wid = lax.axis_index("s") * nc + lax.axis_index("c")
    n_per_w = tokens_hbm.shape[0] // NW
    nch = n_per_w // CH
    base = wid * n_per_w

    pltpu.sync_copy(positions_hbm, pos_v)
    pltpu.sync_copy(gamma_hbm, gb_v.at[0])
    pltpu.sync_copy(beta_hbm, gb_v.at[1])

    lanes = lax.iota(jnp.int32, L)
    perms = [lanes ^ sh for sh in (8, 4, 2, 1)]

    def allsum(v):
        for p in perms:
            v = v + v.at[p].get(mode="promise_in_bounds")
        return v

    def compute_chunk(c):
        def row(j, carry):
            p = lax.rem(c * CH + j, SEQ)
            x = [rows_v[j, pl.ds(k * L, L)] + pos_v[p, pl.ds(k * L, L)]
                 for k in range(NVH)]
            s = x[0] + x[1]
            for k in range(2, NVH):
                s = s + x[k]
            q = x[0] * x[0]
            for k in range(1, NVH):
                q = q + x[k] * x[k]
            mean_v = allsum(s) * jnp.float32(1.0 / HIDDEN)
            var = allsum(q) * jnp.float32(1.0 / HIDDEN) - mean_v * mean_v
            inv = _rsqrt(var + jnp.float32(EPS))
            for k in range(NVH):
                g = gb_v[0, pl.ds(k * L, L)]
                b = gb_v[1, pl.ds(k * L, L)]
                rows_v[j, pl.ds(k * L, L)] = (x[k] - mean_v) * inv * g + b
            return carry

        lax.fori_loop(0, CH, row, 0, unroll=2)

    def chunk(c, carry):
        tok0 = base + c * CH
        pltpu.sync_copy(tokens_hbm.at[pl.ds(tok0, CH)], idx_v)
        copies = [
            pltpu.make_async_copy(
                words_hbm.at[idx_v.at[pl.ds(g * GSUB, GSUB)]],
                rows_v.at[pl.ds(g * GSUB, GSUB)],
                gsem,
            )
            for g in range(CH // GSUB)
        ]
        for cp in copies:
            cp.start()
        for cp in copies:
            cp.wait()
        compute_chunk(c)
        pltpu.make_async_copy(
            rows_v, out_hbm.at[pl.ds(tok0, CH)], osem
        ).start()
        pltpu.make_async_copy(
            rows_v, out_hbm.at[pl.ds(tok0, CH)], osem
        ).wait()
        return carry

    lax.fori_loop(0, nch, chunk, 0)


def kernel(tokens, words, positions, ln_gamma, ln_beta):
    batch, seq = tokens.shape
    n = batch * seq
    tok_flat = tokens.reshape(n).astype(jnp.int32)

    run = functools.partial(
        pl.kernel,
        out_type=jax.ShapeDtypeStruct((n, HIDDEN), jnp.float32),
        mesh=plsc.VectorSubcoreMesh(core_axis_name="c", subcore_axis_name="s"),
        compiler_params=pltpu.CompilerParams(
            needs_layout_passes=False, use_tc_tiling_on_sc=False
        ),
        scratch_types=[
            pltpu.VMEM((CH,), jnp.int32),
            pltpu.VMEM((CH, HIDDEN), jnp.float32),
            pltpu.VMEM((MAX_LEN, HIDDEN), jnp.float32),
            pltpu.VMEM((2, HIDDEN), jnp.float32),
            pltpu.SemaphoreType.DMA,
            pltpu.SemaphoreType.DMA,
        ],
    )(_body)
    out = run(tok_flat, words, positions, ln_gamma, ln_beta)
    return out.reshape(batch, seq, HIDDEN)


# speedup vs baseline: 2.4505x; 1.8349x over previous
import functools

import jax
import jax.numpy as jnp
from jax import lax
from jax.experimental import pallas as pl
from jax.experimental.pallas import tpu as pltpu
from jax.experimental.pallas import tpu_sc as plsc

VOCAB = 1000000
HIDDEN = 64
MAX_LEN = 200
SEQ = 200
EPS = 1e-08

L = 16
NW = 32
CH = 256
GSUB = 128
NVH = HIDDEN // L


def _rsqrt(a):
    i = plsc.bitcast(a, jnp.int32)
    i = jnp.int32(0x5F3759DF) - lax.shift_right_logical(i, 1)
    y = plsc.bitcast(i, jnp.float32)
    half = jnp.float32(0.5) * a
    for _ in range(2):
        y = y * (jnp.float32(1.5) - half * y * y)
    return y


def _body(tokens_hbm, words_hbm, positions_hbm, gamma_hbm, beta_hbm,
          out_hbm, idx0, idx1, rows0, rows1, pos_v, gb_v,
          gsem0, gsem1, osem0, osem1):
    info = plsc.get_sparse_core_info()
    nc = info.num_cores
    wid = lax.axis_index("s") * nc + lax.axis_index("c")
    n_per_w = tokens_hbm.shape[0] // NW
    nch = n_per_w // CH
    base = wid * n_per_w

    bufs = [(idx0, rows0, gsem0, osem0), (idx1, rows1, gsem1, osem1)]

    pltpu.sync_copy(positions_hbm, pos_v)
    pltpu.sync_copy(gamma_hbm, gb_v.at[0])
    pltpu.sync_copy(beta_hbm, gb_v.at[1])

    lanes = lax.iota(jnp.int32, L)
    perms = [lanes ^ sh for sh in (8, 4, 2, 1)]

    def allsum(v):
        for p in perms:
            v = v + v.at[p].get(mode="promise_in_bounds")
        return v

    def gather_copies(c, idx_v, rows_v, gsem):
        return [
            pltpu.make_async_copy(
                words_hbm.at[idx_v.at[pl.ds(g * GSUB, GSUB)]],
                rows_v.at[pl.ds(g * GSUB, GSUB)],
                gsem,
            )
            for g in range(CH // GSUB)
        ]

    def start_gather(c, idx_v, rows_v, gsem):
        pltpu.sync_copy(tokens_hbm.at[pl.ds(base + c * CH, CH)], idx_v)
        for cp in gather_copies(c, idx_v, rows_v, gsem):
            cp.start()

    def out_copy(c, rows_v, osem):
        return pltpu.make_async_copy(
            rows_v, out_hbm.at[pl.ds(base + c * CH, CH)], osem
        )

    def compute_chunk(c, rows_v):
        @plsc.parallel_loop(0, CH, unroll=4)
        def row(j):
            p = lax.rem(c * CH + j, SEQ)
            x = [rows_v[j, pl.ds(k * L, L)] + pos_v[p, pl.ds(k * L, L)]
                 for k in range(NVH)]
            s = (x[0] + x[1]) + (x[2] + x[3])
            q = x[0] * x[0]
            for k in range(1, NVH):
                q = q + x[k] * x[k]
            mean_v = allsum(s) * jnp.float32(1.0 / HIDDEN)
            var = allsum(q) * jnp.float32(1.0 / HIDDEN) - mean_v * mean_v
            inv = _rsqrt(var + jnp.float32(EPS))
            for k in range(NVH):
                g = gb_v[0, pl.ds(k * L, L)]
                b = gb_v[1, pl.ds(k * L, L)]
                rows_v[j, pl.ds(k * L, L)] = (x[k] - mean_v) * inv * g + b

    start_gather(0, idx0, rows0, gsem0)

    def pair(i, carry):
        for b in (0, 1):
            c = 2 * i + b
            o_idx, o_rows, o_gsem, o_osem = bufs[1 - b]
            idx_v, rows_v, gsem, osem = bufs[b]
            if b == 0:
                @pl.when(i > 0)
                def _():
                    out_copy(0, o_rows, o_osem).wait()
                start_gather(c + 1, o_idx, o_rows, o_gsem)
            else:
                @pl.when(i < nch // 2 - 1)
                def _():
                    out_copy(0, o_rows, o_osem).wait()
                    start_gather(c + 1, o_idx, o_rows, o_gsem)
            for cp in gather_copies(c, idx_v, rows_v, gsem):
                cp.wait()
            compute_chunk(c, rows_v)
            out_copy(c, rows_v, osem).start()
        return carry

    lax.fori_loop(0, nch // 2, pair, 0)
    out_copy(0, rows0, osem0).wait()
    out_copy(0, rows1, osem1).wait()


def kernel(tokens, words, positions, ln_gamma, ln_beta):
    batch, seq = tokens.shape
    n = batch * seq
    tok_flat = tokens.reshape(n).astype(jnp.int32)

    run = functools.partial(
        pl.kernel,
        out_type=jax.ShapeDtypeStruct((n, HIDDEN), jnp.float32),
        mesh=plsc.VectorSubcoreMesh(core_axis_name="c", subcore_axis_name="s"),
        compiler_params=pltpu.CompilerParams(
            needs_layout_passes=False, use_tc_tiling_on_sc=False
        ),
        scratch_types=[
            pltpu.VMEM((CH,), jnp.int32),
            pltpu.VMEM((CH,), jnp.int32),
            pltpu.VMEM((CH, HIDDEN), jnp.float32),
            pltpu.VMEM((CH, HIDDEN), jnp.float32),
            pltpu.VMEM((MAX_LEN, HIDDEN), jnp.float32),
            pltpu.VMEM((2, HIDDEN), jnp.float32),
            pltpu.SemaphoreType.DMA,
            pltpu.SemaphoreType.DMA,
            pltpu.SemaphoreType.DMA,
            pltpu.SemaphoreType.DMA,
        ],
    )(_body)
    out = run(tok_flat, words, positions, ln_gamma, ln_beta)
    return out.reshape(batch, seq, HIDDEN)
